# baseline (speedup 1.0000x reference)
import functools

import jax
import jax.numpy as jnp
from jax import lax
from jax.experimental import pallas as pl
from jax.experimental.pallas import tpu as pltpu
from jax.experimental.pallas import tpu_sc as plsc

E = 320000
M = 640000
IC = 16
OC = 16
H = 32
DIM3 = 3

NC = 2
NS = 16
NW = NC * NS
G = 128
IB = 8
SB = 8
MP = 655360
GROWS = MP // G
GPW = GROWS // NW
GPS = GROWS // NS
MB = 1024
NBLK = MP // MB

CHUNK = 80000
PAD = 1920
ACC = CHUNK + PAD
WBB = 200

_mesh = plsc.VectorSubcoreMesh(
    core_axis_name="c", subcore_axis_name="s", num_cores=NC, num_subcores=NS)
_sc_params = pltpu.CompilerParams(use_tc_tiling_on_sc=False)


@functools.partial(
    pl.kernel,
    out_type=jax.ShapeDtypeStruct((MP * IC // 128, 128), jnp.float32),
    mesh=_mesh,
    scratch_types=[
        pltpu.VMEM((IB, G), jnp.int32),
        pltpu.VMEM((IB * G, IC), jnp.float32),
        pltpu.VMEM((IB * G * IC // 128, 128), jnp.float32),
        pltpu.SemaphoreType.DMA,
    ],
    compiler_params=_sc_params,
)
def _sc_gather(ie_hbm, feat_hbm, f_hbm, idx_v, rows_v, out128, sem):
    c = lax.axis_index("c")
    s = lax.axis_index("s")
    wid = s * NC + c
    row0 = wid * GPW
    R128 = IB * G * IC // 128

    def step(t, carry):
        r0 = row0 + t * IB
        pltpu.sync_copy(ie_hbm.at[pl.ds(r0, IB)], idx_v)
        cps = [
            pltpu.async_copy(
                feat_hbm.at[idx_v.at[b]], rows_v.at[pl.ds(b * G, G)], sem)
            for b in range(IB)
        ]
        for cp in cps:
            cp.wait()

        def rep(q, carry2):
            for j in range(8):
                out128[q, pl.ds(j * IC, IC)] = rows_v[j * G + q, :]
            return carry2

        lax.fori_loop(0, R128, rep, 0)
        pltpu.sync_copy(out128, f_hbm.at[pl.ds(r0 * IC, R128)])
        return carry

    lax.fori_loop(0, GPW // IB, step, 0)


def _ln(x, g, b):
    mu = jnp.mean(x, axis=-1, keepdims=True)
    var = jnp.mean((x - mu) ** 2, axis=-1, keepdims=True)
    return (x - mu) / jnp.sqrt(var + 1e-5) * g + b


def _mlp_body(cref, fref, w1, b1, g1, be1, w2, b2, g2, be2, w3, b3, oref):
    x = cref[...]
    f128 = fref[...]
    h = jnp.dot(x, w1[...], preferred_element_type=jnp.float32) + b1[...]
    h = jnp.maximum(_ln(h, g1[...], be1[...]), 0.0)
    h = jnp.dot(h, w2[...], preferred_element_type=jnp.float32) + b2[...]
    h = jnp.maximum(_ln(h, g2[...], be2[...]), 0.0)
    w = jnp.tanh(jnp.dot(h, w3[...], preferred_element_type=jnp.float32) + b3[...])
    jj = lax.broadcasted_iota(jnp.int32, (IC, IC * OC), 1)
    ii = lax.broadcasted_iota(jnp.int32, (IC, IC * OC), 0)
    tmat = (jj % IC == ii).astype(jnp.float32)
    ft = jnp.concatenate(
        [jnp.dot(f128[:, j * IC:(j + 1) * IC], tmat,
                 preferred_element_type=jnp.float32) for j in range(8)],
        axis=0)
    p = ft * w
    qq = lax.broadcasted_iota(jnp.int32, (IC * OC, OC), 0) // IC
    oo = lax.broadcasted_iota(jnp.int32, (IC * OC, OC), 1)
    smat = (qq == oo).astype(jnp.float32)
    out = jnp.dot(p, smat, preferred_element_type=jnp.float32)
    oref[...] = jnp.concatenate(
        [out[j * 128:(j + 1) * 128, :] for j in range(8)], axis=1)


def _rep(shape):
    return pl.BlockSpec(shape, lambda i: tuple(0 for _ in shape))


_mlp_call = pl.pallas_call(
    _mlp_body,
    grid=(NBLK,),
    in_specs=[
        pl.BlockSpec((MB, DIM3), lambda i: (i, 0)),
        pl.BlockSpec((MB * IC // 128, 128), lambda i: (i, 0)),
        _rep((DIM3, H)), _rep((1, H)), _rep((1, H)), _rep((1, H)),
        _rep((H, H)), _rep((1, H)), _rep((1, H)), _rep((1, H)),
        _rep((H, IC * OC)), _rep((1, IC * OC)),
    ],
    out_specs=pl.BlockSpec((MB * OC // 128, 128), lambda i: (i, 0)),
    out_shape=jax.ShapeDtypeStruct((MP * OC // 128, 128), jnp.float32),
)


@functools.partial(
    pl.kernel,
    out_type=[
        jax.ShapeDtypeStruct((E * OC // 128, 128), jnp.float32),
        jax.ShapeDtypeStruct((E * OC // 128, 128), jnp.float32),
    ],
    mesh=_mesh,
    scratch_types=[
        pltpu.VMEM_SHARED((ACC, OC), jnp.float32),
        pltpu.VMEM_SHARED((ACC,), jnp.float32),
        pltpu.VMEM((SB, G), jnp.int32),
        pltpu.VMEM((SB, G), jnp.int32),
        pltpu.VMEM((SB * G * OC // 128, 128), jnp.float32),
        pltpu.VMEM((SB * G, OC), jnp.float32),
        pltpu.VMEM((G,), jnp.float32),
        pltpu.VMEM((WBB, OC), jnp.float32),
        pltpu.VMEM((WBB,), jnp.float32),
        pltpu.VMEM((WBB * OC // 128, 128), jnp.float32),
        pltpu.SemaphoreType.DMA,
    ],
    compiler_params=_sc_params,
)
def _sc_scatter(oe_hbm, rows_hbm, z2_hbm, z1_hbm, sums_hbm, cnt_hbm,
                acc_sp, cnt_sp, idx_v, lidx_v, rows128, rows_v, ones_v,
                wb, wbc, wb128, sem):
    c = lax.axis_index("c")
    s = lax.axis_index("s")
    lanes = lax.iota(jnp.int32, 16)

    for j in range(G // 16):
        ones_v[pl.ds(j * 16, 16)] = jnp.ones((16,), jnp.float32)

    row0 = s * GPS
    STRIPE = ACC // NS

    for k in range(2):
        base = c * (2 * CHUNK) + k * CHUNK

        pltpu.sync_copy(z2_hbm, wb)
        pltpu.sync_copy(z1_hbm, wbc)

        def zstep(z, carry):
            off = s * STRIPE + z * WBB
            pltpu.sync_copy(wb, acc_sp.at[pl.ds(off, WBB)])
            pltpu.sync_copy(wbc, cnt_sp.at[pl.ds(off, WBB)])
            return carry

        lax.fori_loop(0, STRIPE // WBB, zstep, 0)
        ztail = STRIPE - (STRIPE // WBB) * WBB
        zoff = s * STRIPE + (STRIPE // WBB) * WBB
        pltpu.sync_copy(wb.at[pl.ds(0, ztail)], acc_sp.at[pl.ds(zoff, ztail)])
        pltpu.sync_copy(wbc.at[pl.ds(0, ztail)], cnt_sp.at[pl.ds(zoff, ztail)])
        plsc.subcore_barrier()

        def step(t, carry):
            r0 = row0 + t * SB
            pltpu.sync_copy(oe_hbm.at[pl.ds(r0, SB)], idx_v)
            pltpu.sync_copy(rows_hbm.at[pl.ds(r0 * OC, SB * G * OC // 128)],
                            rows128)

            def unp(q, carry2):
                for j in range(8):
                    rows_v[j * G + q, :] = rows128[q, pl.ds(j * OC, OC)]
                return carry2

            lax.fori_loop(0, SB * G * OC // 128, unp, 0)
            for b in range(SB):
                for j in range(G // 16):
                    v = idx_v[b, pl.ds(j * 16, 16)]
                    inb = (v >= base) & (v < base + CHUNK)
                    dummy = CHUNK + ((b * G + j * 16 + lanes) & 1023)
                    lidx_v[b, pl.ds(j * 16, 16)] = jnp.where(
                        inb, v - base, dummy)
            cps = []
            for b in range(SB):
                cps.append(pltpu.async_copy(
                    rows_v.at[pl.ds(b * G, G)],
                    acc_sp.at[lidx_v.at[b]], sem, add=True))
                cps.append(pltpu.async_copy(
                    ones_v, cnt_sp.at[lidx_v.at[b]], sem, add=True))
            for cp in cps:
                cp.wait()
            return carry

        lax.fori_loop(0, GPS // SB, step, 0)
        plsc.subcore_barrier()

        QR = WBB * OC // 128

        def wstep(z, carry):
            off = s * (CHUNK // NS) + z * WBB
            pltpu.sync_copy(acc_sp.at[pl.ds(off, WBB)], wb)
            pltpu.sync_copy(cnt_sp.at[pl.ds(off, WBB)], wbc)
            r0 = (base + off) // 8

            def reps(q, carry2):
                for j in range(8):
                    wb128[q, pl.ds(j * OC, OC)] = wb[j * QR + q, :]
                return carry2

            lax.fori_loop(0, QR, reps, 0)
            pltpu.sync_copy(wb128, sums_hbm.at[pl.ds(r0, QR)])

            def repc(qg, carry2):
                for j in range(8):
                    c16 = wbc[pl.ds(j * QR + qg * 16, 16)]
                    for t in range(16):
                        wb128[qg * 16 + t, pl.ds(j * OC, OC)] = jnp.full(
                            (16,), c16[t], jnp.float32)
                return carry2

            lax.fori_loop(0, QR // 16, repc, 0)
            for j in range(8):
                c16 = wbc[pl.ds(j * QR + QR - 16, 16)]
                for t in range(16):
                    wb128[QR - 16 + t, pl.ds(j * OC, OC)] = jnp.full(
                        (16,), c16[t], jnp.float32)
            pltpu.sync_copy(wb128, cnt_hbm.at[pl.ds(r0, QR)])
            return carry

        lax.fori_loop(0, CHUNK // NS // WBB, wstep, 0)
        plsc.subcore_barrier()


QR = WBB * OC // 128
DRB = 8 * QR
DEB = DRB * 128 // OC


def _unpack(x128):
    return jnp.concatenate(
        [x128[g * QR:(g + 1) * QR, j * OC:(j + 1) * OC]
         for g in range(8) for j in range(8)], axis=0)


def _div_body(sref, cref, oref):
    s = _unpack(sref[...])
    c = _unpack(cref[...])
    oref[...] = s / jnp.maximum(c, 1.0)


_div_call = pl.pallas_call(
    _div_body,
    grid=(E // DEB,),
    in_specs=[
        pl.BlockSpec((DRB, 128), lambda i: (i, 0)),
        pl.BlockSpec((DRB, 128), lambda i: (i, 0)),
    ],
    out_specs=pl.BlockSpec((DEB, OC), lambda i: (i, 0)),
    out_shape=jax.ShapeDtypeStruct((E, OC), jnp.float32),
)


def kernel(in_edges, out_edges, edge_features, hood_coords,
           W1, b1, g1, be1, W2, b2, g2, be2, W3, b3):
    padn = MP - M
    ie = in_edges.astype(jnp.int32)
    oe = out_edges.astype(jnp.int32)
    ie_p = jnp.concatenate(
        [ie, jnp.arange(padn, dtype=jnp.int32) % E]).reshape(GROWS, G)
    oe_p = jnp.concatenate(
        [oe, jnp.full((padn,), -1, jnp.int32)]).reshape(GROWS, G)
    coords_p = jnp.concatenate(
        [hood_coords.astype(jnp.float32),
         jnp.zeros((padn, DIM3), jnp.float32)], axis=0)
    z2 = jnp.zeros((WBB, OC), jnp.float32)
    z1 = jnp.zeros((WBB,), jnp.float32)

    f_p = _sc_gather(ie_p, edge_features.astype(jnp.float32))
    out_p = _mlp_call(
        coords_p, f_p, W1,
        b1.reshape(1, H), g1.reshape(1, H), be1.reshape(1, H),
        W2, b2.reshape(1, H), g2.reshape(1, H), be2.reshape(1, H),
        W3, b3.reshape(1, IC * OC))
    sums, counts = _sc_scatter(oe_p, out_p, z2, z1)
    return _div_call(sums, counts)


# speedup vs baseline: 1.8331x; 1.8331x over previous
import functools

import jax
import jax.numpy as jnp
from jax import lax
from jax.experimental import pallas as pl
from jax.experimental.pallas import tpu as pltpu
from jax.experimental.pallas import tpu_sc as plsc

E = 320000
M = 640000
IC = 16
OC = 16
H = 32
DIM3 = 3

NC = 2
NS = 16
NW = NC * NS
G = 128
IB = 8
SB = 8
MP = 655360
GROWS = MP // G
GPW = GROWS // NW
GPS = GROWS // NS
MB = 1024
NBLK = M // MB

CHUNK = 80000
PAD = 1920
ACC = CHUNK + PAD
WBB = 200

_mesh = plsc.VectorSubcoreMesh(
    core_axis_name="c", subcore_axis_name="s", num_cores=NC, num_subcores=NS)
_sc_params = pltpu.CompilerParams(use_tc_tiling_on_sc=False)


@functools.partial(
    pl.kernel,
    out_type=jax.ShapeDtypeStruct((MP * IC // 128, 128), jnp.float32),
    mesh=_mesh,
    scratch_types=[
        pltpu.VMEM((IB, G), jnp.int32),
        pltpu.VMEM((IB * G, IC), jnp.float32),
        pltpu.VMEM((IB * G * IC // 128, 128), jnp.float32),
        pltpu.SemaphoreType.DMA,
    ],
    compiler_params=_sc_params,
)
def _sc_gather(ie_hbm, feat_hbm, f_hbm, idx_v, rows_v, out128, sem):
    c = lax.axis_index("c")
    s = lax.axis_index("s")
    wid = s * NC + c
    row0 = wid * GPW
    R128 = IB * G * IC // 128

    def step(t, carry):
        r0 = row0 + t * IB
        pltpu.sync_copy(ie_hbm.at[pl.ds(r0, IB)], idx_v)
        cps = [
            pltpu.async_copy(
                feat_hbm.at[idx_v.at[b]], rows_v.at[pl.ds(b * G, G)], sem)
            for b in range(IB)
        ]
        for cp in cps:
            cp.wait()

        def rep(q, carry2):
            for j in range(8):
                out128[q, pl.ds(j * IC, IC)] = rows_v[j * G + q, :]
            return carry2

        lax.fori_loop(0, R128, rep, 0)
        pltpu.sync_copy(out128, f_hbm.at[pl.ds(r0 * IC, R128)])
        return carry

    lax.fori_loop(0, GPW // IB, step, 0)


def _ln(x, g, b):
    mu = jnp.mean(x, axis=-1, keepdims=True)
    var = jnp.mean((x - mu) ** 2, axis=-1, keepdims=True)
    return (x - mu) / jnp.sqrt(var + 1e-5) * g + b


def _mlp_body(cref, fref, w1, b1, g1, be1, w2, b2, g2, be2, w3, b3, oref):
    x = cref[...]
    f128 = fref[...]
    h = jnp.dot(x, w1[...], preferred_element_type=jnp.float32) + b1[...]
    h = jnp.maximum(_ln(h, g1[...], be1[...]), 0.0)
    h = jnp.dot(h, w2[...], preferred_element_type=jnp.float32) + b2[...]
    h = jnp.maximum(_ln(h, g2[...], be2[...]), 0.0)
    w = jnp.tanh(jnp.dot(h, w3[...], preferred_element_type=jnp.float32) + b3[...])
    jj = lax.broadcasted_iota(jnp.int32, (IC, IC * OC), 1)
    ii = lax.broadcasted_iota(jnp.int32, (IC, IC * OC), 0)
    tmat = (jj % IC == ii).astype(jnp.float32)
    ft = jnp.concatenate(
        [jnp.dot(f128[:, j * IC:(j + 1) * IC], tmat,
                 preferred_element_type=jnp.float32) for j in range(8)],
        axis=0)
    p = ft * w
    qq = lax.broadcasted_iota(jnp.int32, (IC * OC, OC), 0) // IC
    oo = lax.broadcasted_iota(jnp.int32, (IC * OC, OC), 1)
    smat = (qq == oo).astype(jnp.float32)
    out = jnp.dot(p, smat, preferred_element_type=jnp.float32)
    oref[...] = jnp.concatenate(
        [out[j * 128:(j + 1) * 128, :] for j in range(8)], axis=1)


def _rep(shape):
    return pl.BlockSpec(shape, lambda i: tuple(0 for _ in shape))


_mlp_call = pl.pallas_call(
    _mlp_body,
    grid=(NBLK,),
    in_specs=[
        pl.BlockSpec((MB, DIM3), lambda i: (i, 0)),
        pl.BlockSpec((MB * IC // 128, 128), lambda i: (i, 0)),
        _rep((DIM3, H)), _rep((1, H)), _rep((1, H)), _rep((1, H)),
        _rep((H, H)), _rep((1, H)), _rep((1, H)), _rep((1, H)),
        _rep((H, IC * OC)), _rep((1, IC * OC)),
    ],
    out_specs=pl.BlockSpec((MB * OC // 128, 128), lambda i: (i, 0)),
    out_shape=jax.ShapeDtypeStruct((MP * OC // 128, 128), jnp.float32),
)


@functools.partial(
    pl.kernel,
    out_type=[
        jax.ShapeDtypeStruct((E * OC // 128, 128), jnp.float32),
        jax.ShapeDtypeStruct((E * OC // 128, 128), jnp.float32),
    ],
    mesh=_mesh,
    scratch_types=[
        pltpu.VMEM_SHARED((ACC, OC), jnp.float32),
        pltpu.VMEM_SHARED((ACC,), jnp.float32),
        pltpu.VMEM((SB, G), jnp.int32),
        pltpu.VMEM((SB, G), jnp.int32),
        pltpu.VMEM((SB * G * OC // 128, 128), jnp.float32),
        pltpu.VMEM((SB * G, OC), jnp.float32),
        pltpu.VMEM((G,), jnp.float32),
        pltpu.VMEM((WBB, OC), jnp.float32),
        pltpu.VMEM((WBB,), jnp.float32),
        pltpu.VMEM((WBB * OC // 128, 128), jnp.float32),
        pltpu.SemaphoreType.DMA,
    ],
    compiler_params=_sc_params,
)
def _sc_scatter(oe_hbm, rows_hbm, z2_hbm, z1_hbm, sums_hbm, cnt_hbm,
                acc_sp, cnt_sp, idx_v, lidx_v, rows128, rows_v, ones_v,
                wb, wbc, wb128, sem):
    c = lax.axis_index("c")
    s = lax.axis_index("s")
    lanes = lax.iota(jnp.int32, 16)

    for j in range(G // 16):
        ones_v[pl.ds(j * 16, 16)] = jnp.ones((16,), jnp.float32)

    row0 = s * GPS
    STRIPE = ACC // NS

    for k in range(2):
        base = c * (2 * CHUNK) + k * CHUNK

        pltpu.sync_copy(z2_hbm, wb)
        pltpu.sync_copy(z1_hbm, wbc)

        def zstep(z, carry):
            off = s * STRIPE + z * WBB
            pltpu.sync_copy(wb, acc_sp.at[pl.ds(off, WBB)])
            pltpu.sync_copy(wbc, cnt_sp.at[pl.ds(off, WBB)])
            return carry

        lax.fori_loop(0, STRIPE // WBB, zstep, 0)
        ztail = STRIPE - (STRIPE // WBB) * WBB
        zoff = s * STRIPE + (STRIPE // WBB) * WBB
        pltpu.sync_copy(wb.at[pl.ds(0, ztail)], acc_sp.at[pl.ds(zoff, ztail)])
        pltpu.sync_copy(wbc.at[pl.ds(0, ztail)], cnt_sp.at[pl.ds(zoff, ztail)])
        plsc.subcore_barrier()

        def step(t, carry):
            r0 = row0 + t * SB
            pltpu.sync_copy(oe_hbm.at[pl.ds(r0, SB)], idx_v)
            pltpu.sync_copy(rows_hbm.at[pl.ds(r0 * OC, SB * G * OC // 128)],
                            rows128)

            def unp(q, carry2):
                for j in range(8):
                    rows_v[j * G + q, :] = rows128[q, pl.ds(j * OC, OC)]
                return carry2

            lax.fori_loop(0, SB * G * OC // 128, unp, 0)
            for b in range(SB):
                for j in range(G // 16):
                    v = idx_v[b, pl.ds(j * 16, 16)]
                    inb = (v >= base) & (v < base + CHUNK)
                    dummy = CHUNK + ((b * G + j * 16 + lanes) & 1023)
                    lidx_v[b, pl.ds(j * 16, 16)] = jnp.where(
                        inb, v - base, dummy)
            cps = []
            for b in range(SB):
                cps.append(pltpu.async_copy(
                    rows_v.at[pl.ds(b * G, G)],
                    acc_sp.at[lidx_v.at[b]], sem, add=True))
                cps.append(pltpu.async_copy(
                    ones_v, cnt_sp.at[lidx_v.at[b]], sem, add=True))
            for cp in cps:
                cp.wait()
            return carry

        lax.fori_loop(0, GPS // SB, step, 0)
        plsc.subcore_barrier()

        QR = WBB * OC // 128

        def wstep(z, carry):
            off = s * (CHUNK // NS) + z * WBB
            pltpu.sync_copy(acc_sp.at[pl.ds(off, WBB)], wb)
            pltpu.sync_copy(cnt_sp.at[pl.ds(off, WBB)], wbc)
            r0 = (base + off) // 8

            def reps(q, carry2):
                for j in range(8):
                    wb128[q, pl.ds(j * OC, OC)] = wb[j * QR + q, :]
                return carry2

            lax.fori_loop(0, QR, reps, 0)
            pltpu.sync_copy(wb128, sums_hbm.at[pl.ds(r0, QR)])

            def repc(qg, carry2):
                for j in range(8):
                    c16 = wbc[pl.ds(j * QR + qg * 16, 16)]
                    for t in range(16):
                        wb128[qg * 16 + t, pl.ds(j * OC, OC)] = jnp.full(
                            (16,), c16[t], jnp.float32)
                return carry2

            lax.fori_loop(0, QR // 16, repc, 0)
            for j in range(8):
                c16 = wbc[pl.ds(j * QR + QR - 16, 16)]
                for t in range(16):
                    wb128[QR - 16 + t, pl.ds(j * OC, OC)] = jnp.full(
                        (16,), c16[t], jnp.float32)
            pltpu.sync_copy(wb128, cnt_hbm.at[pl.ds(r0, QR)])
            return carry

        lax.fori_loop(0, CHUNK // NS // WBB, wstep, 0)
        plsc.subcore_barrier()


QR = WBB * OC // 128
DRB = 8 * QR
DEB = DRB * 128 // OC


def _unpack(x128):
    return jnp.concatenate(
        [x128[g * QR:(g + 1) * QR, j * OC:(j + 1) * OC]
         for g in range(8) for j in range(8)], axis=0)


def _div_body(sref, cref, oref):
    s = _unpack(sref[...])
    c = _unpack(cref[...])
    oref[...] = s / jnp.maximum(c, 1.0)


_div_call = pl.pallas_call(
    _div_body,
    grid=(E // DEB,),
    in_specs=[
        pl.BlockSpec((DRB, 128), lambda i: (i, 0)),
        pl.BlockSpec((DRB, 128), lambda i: (i, 0)),
    ],
    out_specs=pl.BlockSpec((DEB, OC), lambda i: (i, 0)),
    out_shape=jax.ShapeDtypeStruct((E, OC), jnp.float32),
)


def kernel(in_edges, out_edges, edge_features, hood_coords,
           W1, b1, g1, be1, W2, b2, g2, be2, W3, b3):
    padn = MP - M
    ie = in_edges.astype(jnp.int32)
    oe = out_edges.astype(jnp.int32)
    ie_p = jnp.concatenate(
        [ie, jnp.arange(padn, dtype=jnp.int32) % E]).reshape(GROWS, G)
    oe_p = jnp.concatenate(
        [oe, jnp.full((padn,), -1, jnp.int32)]).reshape(GROWS, G)
    z2 = jnp.zeros((WBB, OC), jnp.float32)
    z1 = jnp.zeros((WBB,), jnp.float32)

    f_p = _sc_gather(ie_p, edge_features.astype(jnp.float32))
    out_p = _mlp_call(
        hood_coords.astype(jnp.float32), f_p, W1,
        b1.reshape(1, H), g1.reshape(1, H), be1.reshape(1, H),
        W2, b2.reshape(1, H), g2.reshape(1, H), be2.reshape(1, H),
        W3, b3.reshape(1, IC * OC))
    sums, counts = _sc_scatter(oe_p, out_p, z2, z1)
    return _div_call(sums, counts)


# speedup vs baseline: 1.9456x; 1.0614x over previous
import functools

import jax
import jax.numpy as jnp
from jax import lax
from jax.experimental import pallas as pl
from jax.experimental.pallas import tpu as pltpu
from jax.experimental.pallas import tpu_sc as plsc

E = 320000
M = 640000
IC = 16
OC = 16
H = 32
DIM3 = 3

NC = 2
NS = 16
NW = NC * NS
G = 128
IB = 8
SB = 8
MP = 655360
GROWS = MP // G
GPW = GROWS // NW
GPS = GROWS // NS
MB = 1024
NBLK = M // MB

CHUNK = 80000
PAD = 1920
ACC = CHUNK + PAD
WBB = 200

_mesh = plsc.VectorSubcoreMesh(
    core_axis_name="c", subcore_axis_name="s", num_cores=NC, num_subcores=NS)
_sc_params = pltpu.CompilerParams(use_tc_tiling_on_sc=False)


@functools.partial(
    pl.kernel,
    out_type=jax.ShapeDtypeStruct((MP * IC // 128, 128), jnp.float32),
    mesh=_mesh,
    scratch_types=[
        pltpu.VMEM((IB, G), jnp.int32),
        pltpu.VMEM((IB * G, IC), jnp.float32),
        pltpu.VMEM((IB * G * IC // 128, 128), jnp.float32),
        pltpu.SemaphoreType.DMA,
    ],
    compiler_params=_sc_params,
)
def _sc_gather(ie_hbm, feat_hbm, f_hbm, idx_v, rows_v, out128, sem):
    c = lax.axis_index("c")
    s = lax.axis_index("s")
    wid = s * NC + c
    row0 = wid * GPW
    R128 = IB * G * IC // 128

    def step(t, carry):
        r0 = row0 + t * IB
        pltpu.sync_copy(ie_hbm.at[pl.ds(r0, IB)], idx_v)
        cps = [
            pltpu.async_copy(
                feat_hbm.at[idx_v.at[b]], rows_v.at[pl.ds(b * G, G)], sem)
            for b in range(IB)
        ]
        for cp in cps:
            cp.wait()

        def rep(q, carry2):
            for j in range(8):
                out128[q, pl.ds(j * IC, IC)] = rows_v[j * G + q, :]
            return carry2

        lax.fori_loop(0, R128, rep, 0)
        pltpu.sync_copy(out128, f_hbm.at[pl.ds(r0 * IC, R128)])
        return carry

    lax.fori_loop(0, GPW // IB, step, 0)


def _ln(x, g, b):
    mu = jnp.mean(x, axis=-1, keepdims=True)
    var = jnp.mean((x - mu) ** 2, axis=-1, keepdims=True)
    return (x - mu) / jnp.sqrt(var + 1e-5) * g + b


def _mlp_body(cref, fref, w1, b1, g1, be1, w2, b2, g2, be2, w3, b3, oref):
    x = cref[...]
    f128 = fref[...]
    h = jnp.dot(x, w1[...], preferred_element_type=jnp.float32) + b1[...]
    h = jnp.maximum(_ln(h, g1[...], be1[...]), 0.0)
    h = jnp.dot(h, w2[...], preferred_element_type=jnp.float32) + b2[...]
    h = jnp.maximum(_ln(h, g2[...], be2[...]), 0.0)
    w = jnp.tanh(jnp.dot(h, w3[...], preferred_element_type=jnp.float32) + b3[...])
    jj = lax.broadcasted_iota(jnp.int32, (IC, IC * OC), 1)
    ii = lax.broadcasted_iota(jnp.int32, (IC, IC * OC), 0)
    tmat = (jj % IC == ii).astype(jnp.float32)
    ft = jnp.concatenate(
        [jnp.dot(f128[:, j * IC:(j + 1) * IC], tmat,
                 preferred_element_type=jnp.float32) for j in range(8)],
        axis=0)
    p = ft * w
    qq = lax.broadcasted_iota(jnp.int32, (IC * OC, OC), 0) // IC
    oo = lax.broadcasted_iota(jnp.int32, (IC * OC, OC), 1)
    smat = (qq == oo).astype(jnp.float32)
    oref[...] = jnp.dot(p, smat, preferred_element_type=jnp.float32)


def _rep(shape):
    return pl.BlockSpec(shape, lambda i: tuple(0 for _ in shape))


_mlp_call = pl.pallas_call(
    _mlp_body,
    grid=(NBLK,),
    in_specs=[
        pl.BlockSpec((MB, DIM3), lambda i: (i, 0)),
        pl.BlockSpec((MB * IC // 128, 128), lambda i: (i, 0)),
        _rep((DIM3, H)), _rep((1, H)), _rep((1, H)), _rep((1, H)),
        _rep((H, H)), _rep((1, H)), _rep((1, H)), _rep((1, H)),
        _rep((H, IC * OC)), _rep((1, IC * OC)),
    ],
    out_specs=pl.BlockSpec((MB, OC), lambda i: (i, 0)),
    out_shape=jax.ShapeDtypeStruct((MP, OC), jnp.float32),
)


@functools.partial(
    pl.kernel,
    out_type=[
        jax.ShapeDtypeStruct((E * OC // 128, 128), jnp.float32),
        jax.ShapeDtypeStruct((E * OC // 128, 128), jnp.float32),
    ],
    mesh=_mesh,
    scratch_types=[
        pltpu.VMEM_SHARED((ACC, OC), jnp.float32),
        pltpu.VMEM_SHARED((ACC,), jnp.float32),
        pltpu.VMEM((SB, G), jnp.int32),
        pltpu.VMEM((SB, G), jnp.int32),
        pltpu.VMEM((SB * G, OC), jnp.float32),
        pltpu.VMEM((G,), jnp.float32),
        pltpu.VMEM((WBB, OC), jnp.float32),
        pltpu.VMEM((WBB,), jnp.float32),
        pltpu.VMEM((WBB * OC // 128, 128), jnp.float32),
        pltpu.SemaphoreType.DMA,
    ],
    compiler_params=_sc_params,
)
def _sc_scatter(oe_hbm, rows_hbm, z2_hbm, z1_hbm, sums_hbm, cnt_hbm,
                acc_sp, cnt_sp, idx_v, lidx_v, rows_v, ones_v,
                wb, wbc, wb128, sem):
    c = lax.axis_index("c")
    s = lax.axis_index("s")
    lanes = lax.iota(jnp.int32, 16)

    for j in range(G // 16):
        ones_v[pl.ds(j * 16, 16)] = jnp.ones((16,), jnp.float32)

    row0 = s * GPS
    STRIPE = ACC // NS

    for k in range(2):
        base = c * (2 * CHUNK) + k * CHUNK

        pltpu.sync_copy(z2_hbm, wb)
        pltpu.sync_copy(z1_hbm, wbc)

        def zstep(z, carry):
            off = s * STRIPE + z * WBB
            pltpu.sync_copy(wb, acc_sp.at[pl.ds(off, WBB)])
            pltpu.sync_copy(wbc, cnt_sp.at[pl.ds(off, WBB)])
            return carry

        lax.fori_loop(0, STRIPE // WBB, zstep, 0)
        ztail = STRIPE - (STRIPE // WBB) * WBB
        zoff = s * STRIPE + (STRIPE // WBB) * WBB
        pltpu.sync_copy(wb.at[pl.ds(0, ztail)], acc_sp.at[pl.ds(zoff, ztail)])
        pltpu.sync_copy(wbc.at[pl.ds(0, ztail)], cnt_sp.at[pl.ds(zoff, ztail)])
        plsc.subcore_barrier()

        def step(t, carry):
            r0 = row0 + t * SB
            pltpu.sync_copy(oe_hbm.at[pl.ds(r0, SB)], idx_v)
            pltpu.sync_copy(rows_hbm.at[pl.ds(r0 * G, SB * G)], rows_v)
            for b in range(SB):
                for j in range(G // 16):
                    v = idx_v[b, pl.ds(j * 16, 16)]
                    inb = (v >= base) & (v < base + CHUNK)
                    dummy = CHUNK + ((b * G + j * 16 + lanes) & 1023)
                    lidx_v[b, pl.ds(j * 16, 16)] = jnp.where(
                        inb, v - base, dummy)
            cps = []
            for b in range(SB):
                cps.append(pltpu.async_copy(
                    rows_v.at[pl.ds(b * G, G)],
                    acc_sp.at[lidx_v.at[b]], sem, add=True))
                cps.append(pltpu.async_copy(
                    ones_v, cnt_sp.at[lidx_v.at[b]], sem, add=True))
            for cp in cps:
                cp.wait()
            return carry

        lax.fori_loop(0, GPS // SB, step, 0)
        plsc.subcore_barrier()

        QR = WBB * OC // 128

        def wstep(z, carry):
            off = s * (CHUNK // NS) + z * WBB
            pltpu.sync_copy(acc_sp.at[pl.ds(off, WBB)], wb)
            pltpu.sync_copy(cnt_sp.at[pl.ds(off, WBB)], wbc)
            r0 = (base + off) // 8

            def reps(q, carry2):
                for j in range(8):
                    wb128[q, pl.ds(j * OC, OC)] = wb[j * QR + q, :]
                return carry2

            lax.fori_loop(0, QR, reps, 0)
            pltpu.sync_copy(wb128, sums_hbm.at[pl.ds(r0, QR)])

            def repc(qg, carry2):
                for j in range(8):
                    c16 = wbc[pl.ds(j * QR + qg * 16, 16)]
                    for t in range(16):
                        wb128[qg * 16 + t, pl.ds(j * OC, OC)] = jnp.full(
                            (16,), c16[t], jnp.float32)
                return carry2

            lax.fori_loop(0, QR // 16, repc, 0)
            for j in range(8):
                c16 = wbc[pl.ds(j * QR + QR - 16, 16)]
                for t in range(16):
                    wb128[QR - 16 + t, pl.ds(j * OC, OC)] = jnp.full(
                        (16,), c16[t], jnp.float32)
            pltpu.sync_copy(wb128, cnt_hbm.at[pl.ds(r0, QR)])
            return carry

        lax.fori_loop(0, CHUNK // NS // WBB, wstep, 0)
        plsc.subcore_barrier()


QR = WBB * OC // 128
DRB = 8 * QR
DEB = DRB * 128 // OC


def _unpack(x128):
    return jnp.concatenate(
        [x128[g * QR:(g + 1) * QR, j * OC:(j + 1) * OC]
         for g in range(8) for j in range(8)], axis=0)


def _div_body(sref, cref, oref):
    s = _unpack(sref[...])
    c = _unpack(cref[...])
    oref[...] = s / jnp.maximum(c, 1.0)


_div_call = pl.pallas_call(
    _div_body,
    grid=(E // DEB,),
    in_specs=[
        pl.BlockSpec((DRB, 128), lambda i: (i, 0)),
        pl.BlockSpec((DRB, 128), lambda i: (i, 0)),
    ],
    out_specs=pl.BlockSpec((DEB, OC), lambda i: (i, 0)),
    out_shape=jax.ShapeDtypeStruct((E, OC), jnp.float32),
)


def kernel(in_edges, out_edges, edge_features, hood_coords,
           W1, b1, g1, be1, W2, b2, g2, be2, W3, b3):
    padn = MP - M
    ie = in_edges.astype(jnp.int32)
    oe = out_edges.astype(jnp.int32)
    ie_p = jnp.concatenate(
        [ie, jnp.arange(padn, dtype=jnp.int32) % E]).reshape(GROWS, G)
    oe_p = jnp.concatenate(
        [oe, jnp.full((padn,), -1, jnp.int32)]).reshape(GROWS, G)
    z2 = jnp.zeros((WBB, OC), jnp.float32)
    z1 = jnp.zeros((WBB,), jnp.float32)

    f_p = _sc_gather(ie_p, edge_features.astype(jnp.float32))
    out_p = _mlp_call(
        hood_coords.astype(jnp.float32), f_p, W1,
        b1.reshape(1, H), g1.reshape(1, H), be1.reshape(1, H),
        W2, b2.reshape(1, H), g2.reshape(1, H), be2.reshape(1, H),
        W3, b3.reshape(1, IC * OC))
    sums, counts = _sc_scatter(oe_p, out_p, z2, z1)
    return _div_call(sums, counts)


# speedup vs baseline: 2.0176x; 1.0370x over previous
import functools

import jax
import jax.numpy as jnp
from jax import lax
from jax.experimental import pallas as pl
from jax.experimental.pallas import tpu as pltpu
from jax.experimental.pallas import tpu_sc as plsc

E = 320000
M = 640000
IC = 16
OC = 16
H = 32
DIM3 = 3

NC = 2
NS = 16
NW = NC * NS
G = 128
IB = 8
SB = 8
MP = 655360
GROWS = MP // G
GPW = GROWS // NW
GPS = GROWS // NS
MB = 1024
NBLK = M // MB

CHUNK = 80000
PAD = 1920
ACC = CHUNK + PAD
WBB = 200

_mesh = plsc.VectorSubcoreMesh(
    core_axis_name="c", subcore_axis_name="s", num_cores=NC, num_subcores=NS)
_sc_params = pltpu.CompilerParams(use_tc_tiling_on_sc=False)


@functools.partial(
    pl.kernel,
    out_type=jax.ShapeDtypeStruct((MP * IC // 128, 128), jnp.float32),
    mesh=_mesh,
    scratch_types=[
        pltpu.VMEM((IB, G), jnp.int32),
        pltpu.VMEM((IB * G, IC), jnp.float32),
        pltpu.VMEM((IB * G * IC // 128, 128), jnp.float32),
        pltpu.SemaphoreType.DMA,
    ],
    compiler_params=_sc_params,
)
def _sc_gather(ie_hbm, feat_hbm, f_hbm, idx_v, rows_v, out128, sem):
    c = lax.axis_index("c")
    s = lax.axis_index("s")
    wid = s * NC + c
    row0 = wid * GPW
    R128 = IB * G * IC // 128

    def step(t, carry):
        r0 = row0 + t * IB
        pltpu.sync_copy(ie_hbm.at[pl.ds(r0, IB)], idx_v)
        cps = [
            pltpu.async_copy(
                feat_hbm.at[idx_v.at[b]], rows_v.at[pl.ds(b * G, G)], sem)
            for b in range(IB)
        ]
        for cp in cps:
            cp.wait()

        def rep(q, carry2):
            for j in range(8):
                out128[q, pl.ds(j * IC, IC)] = rows_v[j * G + q, :]
            return carry2

        lax.fori_loop(0, R128, rep, 0)
        pltpu.sync_copy(out128, f_hbm.at[pl.ds(r0 * IC, R128)])
        return carry

    lax.fori_loop(0, GPW // IB, step, 0)


def _ln(x, g, b):
    avg = jnp.full((H, H), 1.0 / H, jnp.float32)
    mu = jnp.dot(x, avg, preferred_element_type=jnp.float32)
    d = x - mu
    var = jnp.dot(d * d, avg, preferred_element_type=jnp.float32)
    return d * jax.lax.rsqrt(var + 1e-5) * g + b


def _mlp_body(cref, fref, w1, b1, g1, be1, w2, b2, g2, be2, w3, b3, oref):
    x = cref[...]
    f128 = fref[...]
    h = jnp.dot(x, w1[...], preferred_element_type=jnp.float32) + b1[...]
    h = jnp.maximum(_ln(h, g1[...], be1[...]), 0.0)
    h = jnp.dot(h, w2[...], preferred_element_type=jnp.float32) + b2[...]
    h = jnp.maximum(_ln(h, g2[...], be2[...]), 0.0)
    w = jnp.tanh(jnp.dot(h, w3[...], preferred_element_type=jnp.float32) + b3[...])
    jj = lax.broadcasted_iota(jnp.int32, (IC, IC * OC), 1)
    ii = lax.broadcasted_iota(jnp.int32, (IC, IC * OC), 0)
    tmat = (jj % IC == ii).astype(jnp.float32)
    ft = jnp.concatenate(
        [jnp.dot(f128[:, j * IC:(j + 1) * IC], tmat,
                 preferred_element_type=jnp.float32) for j in range(8)],
        axis=0)
    p = ft * w
    qq = lax.broadcasted_iota(jnp.int32, (IC * OC, OC), 0) // IC
    oo = lax.broadcasted_iota(jnp.int32, (IC * OC, OC), 1)
    smat = (qq == oo).astype(jnp.float32)
    oref[...] = jnp.dot(p, smat, preferred_element_type=jnp.float32)


def _rep(shape):
    return pl.BlockSpec(shape, lambda i: tuple(0 for _ in shape))


_mlp_call = pl.pallas_call(
    _mlp_body,
    grid=(NBLK,),
    in_specs=[
        pl.BlockSpec((MB, DIM3), lambda i: (i, 0)),
        pl.BlockSpec((MB * IC // 128, 128), lambda i: (i, 0)),
        _rep((DIM3, H)), _rep((1, H)), _rep((1, H)), _rep((1, H)),
        _rep((H, H)), _rep((1, H)), _rep((1, H)), _rep((1, H)),
        _rep((H, IC * OC)), _rep((1, IC * OC)),
    ],
    out_specs=pl.BlockSpec((MB, OC), lambda i: (i, 0)),
    out_shape=jax.ShapeDtypeStruct((MP, OC), jnp.float32),
)


@functools.partial(
    pl.kernel,
    out_type=[
        jax.ShapeDtypeStruct((E * OC // 128, 128), jnp.float32),
        jax.ShapeDtypeStruct((E * OC // 128, 128), jnp.float32),
    ],
    mesh=_mesh,
    scratch_types=[
        pltpu.VMEM_SHARED((ACC, OC), jnp.float32),
        pltpu.VMEM_SHARED((ACC,), jnp.float32),
        pltpu.VMEM((SB, G), jnp.int32),
        pltpu.VMEM((SB, G), jnp.int32),
        pltpu.VMEM((SB * G, OC), jnp.float32),
        pltpu.VMEM((G,), jnp.float32),
        pltpu.VMEM((WBB, OC), jnp.float32),
        pltpu.VMEM((WBB,), jnp.float32),
        pltpu.VMEM((WBB * OC // 128, 128), jnp.float32),
        pltpu.SemaphoreType.DMA,
    ],
    compiler_params=_sc_params,
)
def _sc_scatter(oe_hbm, rows_hbm, z2_hbm, z1_hbm, sums_hbm, cnt_hbm,
                acc_sp, cnt_sp, idx_v, lidx_v, rows_v, ones_v,
                wb, wbc, wb128, sem):
    c = lax.axis_index("c")
    s = lax.axis_index("s")
    lanes = lax.iota(jnp.int32, 16)

    for j in range(G // 16):
        ones_v[pl.ds(j * 16, 16)] = jnp.ones((16,), jnp.float32)

    row0 = s * GPS
    STRIPE = ACC // NS

    for k in range(2):
        base = c * (2 * CHUNK) + k * CHUNK

        pltpu.sync_copy(z2_hbm, wb)
        pltpu.sync_copy(z1_hbm, wbc)

        def zstep(z, carry):
            off = s * STRIPE + z * WBB
            pltpu.sync_copy(wb, acc_sp.at[pl.ds(off, WBB)])
            pltpu.sync_copy(wbc, cnt_sp.at[pl.ds(off, WBB)])
            return carry

        lax.fori_loop(0, STRIPE // WBB, zstep, 0)
        ztail = STRIPE - (STRIPE // WBB) * WBB
        zoff = s * STRIPE + (STRIPE // WBB) * WBB
        pltpu.sync_copy(wb.at[pl.ds(0, ztail)], acc_sp.at[pl.ds(zoff, ztail)])
        pltpu.sync_copy(wbc.at[pl.ds(0, ztail)], cnt_sp.at[pl.ds(zoff, ztail)])
        plsc.subcore_barrier()

        def step(t, carry):
            r0 = row0 + t * SB
            pltpu.sync_copy(oe_hbm.at[pl.ds(r0, SB)], idx_v)
            pltpu.sync_copy(rows_hbm.at[pl.ds(r0 * G, SB * G)], rows_v)
            for b in range(SB):
                for j in range(G // 16):
                    v = idx_v[b, pl.ds(j * 16, 16)]
                    inb = (v >= base) & (v < base + CHUNK)
                    dummy = CHUNK + ((b * G + j * 16 + lanes) & 1023)
                    lidx_v[b, pl.ds(j * 16, 16)] = jnp.where(
                        inb, v - base, dummy)
            cps = []
            for b in range(SB):
                cps.append(pltpu.async_copy(
                    rows_v.at[pl.ds(b * G, G)],
                    acc_sp.at[lidx_v.at[b]], sem, add=True))
                cps.append(pltpu.async_copy(
                    ones_v, cnt_sp.at[lidx_v.at[b]], sem, add=True))
            for cp in cps:
                cp.wait()
            return carry

        lax.fori_loop(0, GPS // SB, step, 0)
        plsc.subcore_barrier()

        QR = WBB * OC // 128

        def wstep(z, carry):
            off = s * (CHUNK // NS) + z * WBB
            pltpu.sync_copy(acc_sp.at[pl.ds(off, WBB)], wb)
            pltpu.sync_copy(cnt_sp.at[pl.ds(off, WBB)], wbc)
            r0 = (base + off) // 8

            def reps(q, carry2):
                for j in range(8):
                    wb128[q, pl.ds(j * OC, OC)] = wb[j * QR + q, :]
                return carry2

            lax.fori_loop(0, QR, reps, 0)
            pltpu.sync_copy(wb128, sums_hbm.at[pl.ds(r0, QR)])

            def repc(qg, carry2):
                for j in range(8):
                    c16 = wbc[pl.ds(j * QR + qg * 16, 16)]
                    for t in range(16):
                        wb128[qg * 16 + t, pl.ds(j * OC, OC)] = jnp.full(
                            (16,), c16[t], jnp.float32)
                return carry2

            lax.fori_loop(0, QR // 16, repc, 0)
            for j in range(8):
                c16 = wbc[pl.ds(j * QR + QR - 16, 16)]
                for t in range(16):
                    wb128[QR - 16 + t, pl.ds(j * OC, OC)] = jnp.full(
                        (16,), c16[t], jnp.float32)
            pltpu.sync_copy(wb128, cnt_hbm.at[pl.ds(r0, QR)])
            return carry

        lax.fori_loop(0, CHUNK // NS // WBB, wstep, 0)
        plsc.subcore_barrier()


QR = WBB * OC // 128
DRB = 8 * QR
DEB = DRB * 128 // OC


def _unpack(x128):
    return jnp.concatenate(
        [x128[g * QR:(g + 1) * QR, j * OC:(j + 1) * OC]
         for g in range(8) for j in range(8)], axis=0)


def _div_body(sref, cref, oref):
    s = _unpack(sref[...])
    c = _unpack(cref[...])
    oref[...] = s / jnp.maximum(c, 1.0)


_div_call = pl.pallas_call(
    _div_body,
    grid=(E // DEB,),
    in_specs=[
        pl.BlockSpec((DRB, 128), lambda i: (i, 0)),
        pl.BlockSpec((DRB, 128), lambda i: (i, 0)),
    ],
    out_specs=pl.BlockSpec((DEB, OC), lambda i: (i, 0)),
    out_shape=jax.ShapeDtypeStruct((E, OC), jnp.float32),
)


def kernel(in_edges, out_edges, edge_features, hood_coords,
           W1, b1, g1, be1, W2, b2, g2, be2, W3, b3):
    padn = MP - M
    ie = in_edges.astype(jnp.int32)
    oe = out_edges.astype(jnp.int32)
    ie_p = jnp.concatenate(
        [ie, jnp.arange(padn, dtype=jnp.int32) % E]).reshape(GROWS, G)
    oe_p = jnp.concatenate(
        [oe, jnp.full((padn,), -1, jnp.int32)]).reshape(GROWS, G)
    z2 = jnp.zeros((WBB, OC), jnp.float32)
    z1 = jnp.zeros((WBB,), jnp.float32)

    f_p = _sc_gather(ie_p, edge_features.astype(jnp.float32))
    out_p = _mlp_call(
        hood_coords.astype(jnp.float32), f_p, W1,
        b1.reshape(1, H), g1.reshape(1, H), be1.reshape(1, H),
        W2, b2.reshape(1, H), g2.reshape(1, H), be2.reshape(1, H),
        W3, b3.reshape(1, IC * OC))
    sums, counts = _sc_scatter(oe_p, out_p, z2, z1)
    return _div_call(sums, counts)
